# Initial kernel scaffold; baseline (speedup 1.0000x reference)
import jax
import jax.numpy as jnp
from jax.experimental import pallas as pl


def kernel(x_nchw, w1, b1, w2, b2, w3, b3, w4, b4, w5, b5):
    raise NotImplementedError("write your pallas kernel here")


# speedup vs baseline: 1.6535x; 1.6535x over previous
import functools

import jax
import jax.numpy as jnp
from jax.experimental import pallas as pl
from jax.experimental.pallas import tpu as pltpu


def _net1_body(x_ref, w1_ref, b1_ref, w2_ref, b2_ref, w3_ref, b3_ref,
               w4_ref, b4_ref, w5_ref, b5_ref, o_ref):
    bt = x_ref.shape[0]
    f32 = jnp.float32
    bf16 = jnp.bfloat16

    x = x_ref[...].reshape(bt, 28, 28)
    xp = jnp.concatenate([x, jnp.zeros((bt, 8, 28), f32)], axis=1)
    xp = jnp.concatenate([xp, jnp.zeros((bt, 36, 8), f32)], axis=2)
    X4 = xp.reshape(bt, 9, 4, 36)
    slabs = []
    for pr in range(2):
        for ps in range(2):
            taps = []
            for u in range(5):
                c, j = divmod(2 * pr + u, 4)
                taps.append(X4[:, c:c + 8, j, 2 * ps:2 * ps + 32])
            slabs.append(jnp.concatenate(taps, axis=-1))
    p1 = jnp.stack(slabs, axis=1).reshape(bt * 32, 160).astype(bf16)

    h1 = jnp.dot(p1, w1_ref[...], preferred_element_type=f32)
    h1 = jnp.maximum(jnp.maximum(h1[:, 0:128], h1[:, 128:256]),
                     jnp.maximum(h1[:, 256:384], h1[:, 384:512]))
    y1 = jnp.maximum(h1 + b1_ref[...], 0.0)
    y1 = y1.reshape(bt, 4, 8, 128).astype(bf16)

    pieces = []
    for Dr in range(5):
        t = Dr // 2
        for ps in range(2):
            q = (Dr % 2) * 2 + ps
            blk = y1[:, q]
            if t > 0:
                blk = jnp.concatenate(
                    [blk[:, t:, :], jnp.zeros((bt, t, 128), blk.dtype)], axis=1)
            pieces.append(blk)
    p2 = jnp.concatenate(pieces, axis=-1).reshape(bt * 8, 1280)

    h2 = jnp.dot(p2, w2_ref[...], preferred_element_type=f32)
    h2 = jnp.maximum(h2 + b2_ref[...], 0.0)
    y2 = 0.25 * (h2[:, 0:160] + h2[:, 160:320] +
                 h2[:, 320:480] + h2[:, 480:640])

    y2 = y2.reshape(bt, 8, 160)
    p3 = jnp.concatenate([y2[:, R, :] for R in range(5)],
                         axis=-1).astype(bf16)
    h3 = jnp.maximum(jnp.dot(p3, w3_ref[...], preferred_element_type=f32)
                     + b3_ref[...], 0.0)

    h4 = jnp.tanh(jnp.dot(h3, w4_ref[...], preferred_element_type=f32)
                  + b4_ref[...])
    lg = jnp.dot(h4, w5_ref[...], preferred_element_type=f32) + b5_ref[...]
    m = jnp.max(lg, axis=-1, keepdims=True)
    e = jnp.exp(lg - m)
    s = jnp.sum(e, axis=-1, keepdims=True)
    o_ref[...] = (e / s)[None]


@functools.partial(jax.jit, static_argnames=("block_b",))
def _net1_forward(x_nchw, w1, b1, w2, b2, w3, b3, w4, b4, w5, b5, block_b=32):
    B = x_nchw.shape[0]
    nb = B // block_b
    w1c = w1.astype(jnp.bfloat16)
    w2c = w2.astype(jnp.bfloat16)
    w3c = w3.astype(jnp.bfloat16)

    out = pl.pallas_call(
        _net1_body,
        out_shape=jax.ShapeDtypeStruct((nb, block_b, 128), jnp.float32),
        grid=(nb,),
        in_specs=[
            pl.BlockSpec((block_b, 1, 28, 28), lambda i: (i, 0, 0, 0)),
            pl.BlockSpec((160, 512), lambda i: (0, 0)),
            pl.BlockSpec((1, 128), lambda i: (0, 0)),
            pl.BlockSpec((1280, 640), lambda i: (0, 0)),
            pl.BlockSpec((1, 640), lambda i: (0, 0)),
            pl.BlockSpec((800, 64), lambda i: (0, 0)),
            pl.BlockSpec((1, 64), lambda i: (0, 0)),
            pl.BlockSpec((64, 32), lambda i: (0, 0)),
            pl.BlockSpec((1, 32), lambda i: (0, 0)),
            pl.BlockSpec((32, 128), lambda i: (0, 0)),
            pl.BlockSpec((1, 128), lambda i: (0, 0)),
        ],
        out_specs=pl.BlockSpec((1, block_b, 128), lambda i: (i, 0, 0)),
        compiler_params=pltpu.CompilerParams(
            dimension_semantics=("parallel",),
            vmem_limit_bytes=64 * 1024 * 1024),
    )(x_nchw, w1c, b1, w2c, b2, w3c, b3, w4, b4, w5, b5)
    return out.reshape(B, 128)[:, :10]


def kernel(x_nchw, w1, b1, w2, b2, w3, b3, w4, b4, w5, b5):
    B = x_nchw.shape[0]
    block_b = 32 if B % 32 == 0 else 1
    return _net1_forward(x_nchw, w1, b1, w2, b2, w3, b3, w4, b4, w5, b5,
                         block_b=block_b)


# speedup vs baseline: 3.2271x; 1.9516x over previous
import functools

import jax
import jax.numpy as jnp
from jax.experimental import pallas as pl
from jax.experimental.pallas import tpu as pltpu


def _net1_body(x_ref, w1_ref, b1_ref, w2_ref, b2_ref, w3_ref, b3_ref,
               w4_ref, b4_ref, w5_ref, b5_ref, o_ref):
    bt = x_ref.shape[0]
    f32 = jnp.float32
    bf16 = jnp.bfloat16

    x = jnp.transpose(x_ref[...].reshape(bt, 28, 28).astype(bf16), (1, 0, 2))
    xp = jnp.concatenate([x, jnp.zeros((8, bt, 28), bf16)], axis=0)
    xp = jnp.concatenate([xp, jnp.zeros((36, bt, 8), bf16)], axis=2)
    X4 = xp.reshape(9, 4, bt, 36)

    P = [X4[:8, 0], X4[:8, 1], X4[:8, 2], X4[:8, 3],
         X4[1:9, 0], X4[1:9, 1], X4[1:9, 2]]
    wins = [jnp.concatenate([P[2 * pr + u] for u in range(5)], axis=-1)
            for pr in range(2)]
    p1 = jnp.stack(wins, axis=0).reshape(16 * bt, 180)

    h1 = jnp.dot(p1, w1_ref[...], preferred_element_type=f32)
    h1 = h1.reshape(2, 8, bt, 1024)
    b1v = b1_ref[...]
    y1 = []
    for pr in range(2):
        for ps in range(2):
            o = ps * 512
            m = jnp.maximum(
                jnp.maximum(h1[pr, :, :, o:o + 128], h1[pr, :, :, o + 128:o + 256]),
                jnp.maximum(h1[pr, :, :, o + 256:o + 384], h1[pr, :, :, o + 384:o + 512]))
            y1.append(jnp.maximum(m + b1v, 0.0).astype(bf16))

    pieces = []
    for Dr in range(5):
        t = Dr // 2
        for ps in range(2):
            q = (Dr % 2) * 2 + ps
            pieces.append(y1[q][t:t + 5])
    p2 = jnp.concatenate(pieces, axis=-1).reshape(5 * bt, 1280)

    h2 = jnp.dot(p2, w2_ref[...], preferred_element_type=f32)
    h2 = jnp.maximum(h2 + b2_ref[...], 0.0)
    y2a = (h2[:, 0:128] + h2[:, 128:256]) + (h2[:, 256:384] + h2[:, 384:512])
    y2b = (h2[:, 512:544] + h2[:, 544:576]) + (h2[:, 576:608] + h2[:, 608:640])
    y2 = (0.25 * jnp.concatenate([y2a, y2b], axis=-1)).astype(bf16)
    y2 = y2.reshape(5, bt, 160)

    w3 = w3_ref[...]
    h3 = jnp.dot(y2[0], w3[0:160], preferred_element_type=f32)
    for R in range(1, 5):
        h3 = h3 + jnp.dot(y2[R], w3[160 * R:160 * (R + 1)],
                          preferred_element_type=f32)
    h3 = jnp.maximum(h3 + b3_ref[...], 0.0).astype(bf16)

    h4 = jnp.tanh(jnp.dot(h3, w4_ref[...], preferred_element_type=f32)
                  + b4_ref[...]).astype(bf16)
    lg = jnp.dot(h4, w5_ref[...], preferred_element_type=f32) + b5_ref[...]
    m = jnp.max(lg, axis=-1, keepdims=True)
    e = jnp.exp(lg - m)
    s = jnp.sum(e, axis=-1, keepdims=True)
    o_ref[...] = (e / s)[None]


@functools.partial(jax.jit, static_argnames=("block_b",))
def _net1_forward(x_nchw, w1, b1, w2, b2, w3, b3, w4, b4, w5, b5, block_b=32):
    B = x_nchw.shape[0]
    nb = B // block_b

    w1r = w1.reshape(5, 32, 512)
    W1p = jnp.stack([jnp.pad(w1r, ((0, 0), (0, 4), (0, 0))),
                     jnp.pad(w1r, ((0, 0), (2, 2), (0, 0)))], axis=2)
    W1p = W1p.reshape(180, 1024).astype(jnp.bfloat16)

    w2r = w2.reshape(1280, 4, 160)
    w2p = jnp.concatenate([w2r[:, g, :128] for g in range(4)]
                          + [w2r[:, g, 128:] for g in range(4)], axis=-1)
    w2p = w2p.astype(jnp.bfloat16)
    b2r = b2.reshape(1, 4, 160)
    b2p = jnp.concatenate([b2r[:, g, :128] for g in range(4)]
                          + [b2r[:, g, 128:] for g in range(4)], axis=-1)
    w3c = w3.astype(jnp.bfloat16)
    w4c = w4.astype(jnp.bfloat16)
    w5c = w5.astype(jnp.bfloat16)

    out = pl.pallas_call(
        _net1_body,
        out_shape=jax.ShapeDtypeStruct((nb, block_b, 128), jnp.float32),
        grid=(nb,),
        in_specs=[
            pl.BlockSpec((block_b, 1, 28, 28), lambda i: (i, 0, 0, 0)),
            pl.BlockSpec((180, 1024), lambda i: (0, 0)),
            pl.BlockSpec((1, 128), lambda i: (0, 0)),
            pl.BlockSpec((1280, 640), lambda i: (0, 0)),
            pl.BlockSpec((1, 640), lambda i: (0, 0)),
            pl.BlockSpec((800, 64), lambda i: (0, 0)),
            pl.BlockSpec((1, 64), lambda i: (0, 0)),
            pl.BlockSpec((64, 32), lambda i: (0, 0)),
            pl.BlockSpec((1, 32), lambda i: (0, 0)),
            pl.BlockSpec((32, 128), lambda i: (0, 0)),
            pl.BlockSpec((1, 128), lambda i: (0, 0)),
        ],
        out_specs=pl.BlockSpec((1, block_b, 128), lambda i: (i, 0, 0)),
        compiler_params=pltpu.CompilerParams(
            dimension_semantics=("parallel",),
            vmem_limit_bytes=64 * 1024 * 1024),
    )(x_nchw, W1p, b1, w2p, b2p, w3c, b3, w4c, b4, w5c, b5)
    return out.reshape(B, 128)[:, :10]


def kernel(x_nchw, w1, b1, w2, b2, w3, b3, w4, b4, w5, b5):
    B = x_nchw.shape[0]
    block_b = 128 if B % 128 == 0 else (32 if B % 32 == 0 else 1)
    return _net1_forward(x_nchw, w1, b1, w2, b2, w3, b3, w4, b4, w5, b5,
                         block_b=block_b)


# speedup vs baseline: 4.2028x; 1.3023x over previous
import functools

import jax
import jax.numpy as jnp
from jax.experimental import pallas as pl
from jax.experimental.pallas import tpu as pltpu


def _net1_body(x_ref, w1_ref, b1_ref, w2_ref, b2_ref, w3_ref, b3_ref,
               w4_ref, b4_ref, w5_ref, b5_ref, o_ref):
    bt = x_ref.shape[3]
    f32 = jnp.float32
    bf16 = jnp.bfloat16

    x = jnp.transpose(x_ref[...].reshape(28, 28, bt).astype(bf16), (0, 2, 1))
    xp = jnp.concatenate([x, jnp.zeros((8, bt, 28), bf16)], axis=0)
    xp = jnp.concatenate([xp, jnp.zeros((36, bt, 8), bf16)], axis=2)
    X4 = xp.reshape(9, 4, bt, 36)

    P = [X4[:8, 0], X4[:8, 1], X4[:8, 2], X4[:8, 3],
         X4[1:9, 0], X4[1:9, 1], X4[1:9, 2]]
    wins = [jnp.concatenate([P[2 * pr + u] for u in range(5)], axis=-1)
            for pr in range(2)]
    p1 = jnp.stack(wins, axis=0).reshape(16 * bt, 180)

    h1 = jnp.dot(p1, w1_ref[...], preferred_element_type=f32)
    h1 = h1.reshape(2, 8, bt, 1024)
    b1v = b1_ref[...]
    y1 = []
    for pr in range(2):
        for ps in range(2):
            o = ps * 512
            m = jnp.maximum(
                jnp.maximum(h1[pr, :, :, o:o + 128], h1[pr, :, :, o + 128:o + 256]),
                jnp.maximum(h1[pr, :, :, o + 256:o + 384], h1[pr, :, :, o + 384:o + 512]))
            y1.append(jnp.maximum(m + b1v, 0.0).astype(bf16))

    pieces = []
    for Dr in range(5):
        t = Dr // 2
        for ps in range(2):
            q = (Dr % 2) * 2 + ps
            pieces.append(y1[q][t:t + 5])
    p2 = jnp.concatenate(pieces, axis=-1).reshape(5 * bt, 1280)

    h2 = jnp.dot(p2, w2_ref[...], preferred_element_type=f32)
    h2 = jnp.maximum(h2 + b2_ref[...], 0.0)
    y2a = (h2[:, 0:128] + h2[:, 128:256]) + (h2[:, 256:384] + h2[:, 384:512])
    y2b = (h2[:, 512:544] + h2[:, 544:576]) + (h2[:, 576:608] + h2[:, 608:640])
    y2 = (0.25 * jnp.concatenate([y2a, y2b], axis=-1)).astype(bf16)
    y2 = y2.reshape(5, bt, 160)

    w3 = w3_ref[...]
    h3 = jnp.dot(y2[0], w3[0:160], preferred_element_type=f32)
    for R in range(1, 5):
        h3 = h3 + jnp.dot(y2[R], w3[160 * R:160 * (R + 1)],
                          preferred_element_type=f32)
    h3 = jnp.maximum(h3 + b3_ref[...], 0.0).astype(bf16)

    h4 = jnp.tanh(jnp.dot(h3, w4_ref[...], preferred_element_type=f32)
                  + b4_ref[...]).astype(bf16)
    lg = jnp.dot(h4, w5_ref[...], preferred_element_type=f32) + b5_ref[...]
    m = jnp.max(lg, axis=-1, keepdims=True)
    e = jnp.exp(lg - m)
    s = jnp.sum(e, axis=-1, keepdims=True)
    o_ref[...] = (e / s)[None]


@functools.partial(jax.jit, static_argnames=("block_b",))
def _net1_forward(x_nchw, w1, b1, w2, b2, w3, b3, w4, b4, w5, b5, block_b=32):
    B = x_nchw.shape[0]
    nb = B // block_b
    xt = jnp.transpose(x_nchw, (2, 3, 1, 0))

    w1r = w1.reshape(5, 32, 512)
    W1p = jnp.stack([jnp.pad(w1r, ((0, 0), (0, 4), (0, 0))),
                     jnp.pad(w1r, ((0, 0), (2, 2), (0, 0)))], axis=2)
    W1p = W1p.reshape(180, 1024).astype(jnp.bfloat16)

    w2r = w2.reshape(1280, 4, 160)
    w2p = jnp.concatenate([w2r[:, g, :128] for g in range(4)]
                          + [w2r[:, g, 128:] for g in range(4)], axis=-1)
    w2p = w2p.astype(jnp.bfloat16)
    b2r = b2.reshape(1, 4, 160)
    b2p = jnp.concatenate([b2r[:, g, :128] for g in range(4)]
                          + [b2r[:, g, 128:] for g in range(4)], axis=-1)
    w3c = w3.astype(jnp.bfloat16)
    w4c = w4.astype(jnp.bfloat16)
    w5c = w5.astype(jnp.bfloat16)

    out = pl.pallas_call(
        _net1_body,
        out_shape=jax.ShapeDtypeStruct((nb, block_b, 128), jnp.float32),
        grid=(nb,),
        in_specs=[
            pl.BlockSpec((28, 28, 1, block_b), lambda i: (0, 0, 0, i)),
            pl.BlockSpec((180, 1024), lambda i: (0, 0)),
            pl.BlockSpec((1, 128), lambda i: (0, 0)),
            pl.BlockSpec((1280, 640), lambda i: (0, 0)),
            pl.BlockSpec((1, 640), lambda i: (0, 0)),
            pl.BlockSpec((800, 64), lambda i: (0, 0)),
            pl.BlockSpec((1, 64), lambda i: (0, 0)),
            pl.BlockSpec((64, 32), lambda i: (0, 0)),
            pl.BlockSpec((1, 32), lambda i: (0, 0)),
            pl.BlockSpec((32, 128), lambda i: (0, 0)),
            pl.BlockSpec((1, 128), lambda i: (0, 0)),
        ],
        out_specs=pl.BlockSpec((1, block_b, 128), lambda i: (i, 0, 0)),
        compiler_params=pltpu.CompilerParams(
            dimension_semantics=("parallel",),
            vmem_limit_bytes=64 * 1024 * 1024),
    )(xt, W1p, b1, w2p, b2p, w3c, b3, w4c, b4, w5c, b5)
    return out.reshape(B, 128)[:, :10]


def kernel(x_nchw, w1, b1, w2, b2, w3, b3, w4, b4, w5, b5):
    B = x_nchw.shape[0]
    block_b = 128 if B % 128 == 0 else (32 if B % 32 == 0 else 1)
    return _net1_forward(x_nchw, w1, b1, w2, b2, w3, b3, w4, b4, w5, b5,
                         block_b=block_b)


# speedup vs baseline: 4.6871x; 1.1152x over previous
import functools

import jax
import jax.numpy as jnp
from jax.experimental import pallas as pl
from jax.experimental.pallas import tpu as pltpu


def _net1_body(x_ref, w1_ref, b1_ref, w2_ref, b2_ref, w3_ref, b3_ref,
               w4_ref, b4_ref, w5_ref, b5_ref, o_ref):
    bt = x_ref.shape[3]
    f32 = jnp.float32
    bf16 = jnp.bfloat16

    x = x_ref[...].reshape(28, 28, bt).astype(bf16)
    xp = jnp.concatenate([x, jnp.zeros((28, 12, bt), bf16)], axis=1)
    xp = jnp.concatenate([xp, jnp.zeros((8, 40, bt), bf16)], axis=0)

    p1 = jnp.concatenate(
        [xp[4 * ar + 2 * pr:4 * ar + 2 * pr + 5].reshape(200, bt)
         for pr in range(2) for ar in range(8)], axis=-1)

    h1 = jnp.dot(w1_ref[...], p1, preferred_element_type=f32)
    b1v = b1_ref[...]
    y1 = []
    for ps in range(2):
        o = ps * 512
        m = jnp.maximum(jnp.maximum(h1[o:o + 128], h1[o + 128:o + 256]),
                        jnp.maximum(h1[o + 256:o + 384], h1[o + 384:o + 512]))
        y1.append(jnp.maximum(m + b1v, 0.0).astype(bf16))

    pieces = []
    for Dr in range(5):
        t = Dr // 2
        pr = Dr % 2
        for ps in range(2):
            o = pr * 8 * bt + t * bt
            pieces.append(y1[ps][:, o:o + 5 * bt])
    p2 = jnp.concatenate(pieces, axis=0)

    h2 = jnp.dot(w2_ref[...], p2, preferred_element_type=f32)
    h2 = jnp.maximum(h2 + b2_ref[...], 0.0)
    y2a = (h2[0:128] + h2[128:256]) + (h2[256:384] + h2[384:512])
    y2b = (h2[512:544] + h2[544:576]) + (h2[576:608] + h2[608:640])
    y2 = (0.25 * jnp.concatenate([y2a, y2b], axis=0)).astype(bf16)

    w3 = w3_ref[...]
    h3 = jnp.dot(w3[:, 0:160], y2[:, 0:bt], preferred_element_type=f32)
    for R in range(1, 5):
        h3 = h3 + jnp.dot(w3[:, 160 * R:160 * (R + 1)],
                          y2[:, R * bt:(R + 1) * bt],
                          preferred_element_type=f32)
    h3 = jnp.maximum(h3 + b3_ref[...], 0.0).astype(bf16)

    h4 = jnp.tanh(jnp.dot(w4_ref[...], h3, preferred_element_type=f32)
                  + b4_ref[...]).astype(bf16)
    lg = jnp.dot(w5_ref[...], h4, preferred_element_type=f32) + b5_ref[...]
    m = jnp.max(lg, axis=0, keepdims=True)
    e = jnp.exp(lg - m)
    s = jnp.sum(e, axis=0, keepdims=True)
    o_ref[...] = jnp.transpose(e / s)[None]


@functools.partial(jax.jit, static_argnames=("block_b",))
def _net1_forward(x_nchw, w1, b1, w2, b2, w3, b3, w4, b4, w5, b5, block_b=128):
    B = x_nchw.shape[0]
    nb = B // block_b
    xt = jnp.transpose(x_nchw, (2, 3, 1, 0))

    w1r = w1.reshape(5, 32, 512)
    W1p = jnp.stack([jnp.pad(w1r, ((0, 0), (0, 8), (0, 0))),
                     jnp.pad(w1r, ((0, 0), (2, 6), (0, 0)))], axis=0)
    W1p = jnp.transpose(W1p, (0, 3, 1, 2)).reshape(1024, 200)
    W1p = W1p.astype(jnp.bfloat16)

    w2r = w2.reshape(1280, 4, 160)
    w2p = jnp.concatenate([w2r[:, g, :128] for g in range(4)]
                          + [w2r[:, g, 128:] for g in range(4)], axis=-1)
    W2p = jnp.transpose(w2p).astype(jnp.bfloat16)
    b2r = b2.reshape(1, 4, 160)
    b2p = jnp.concatenate([b2r[:, g, :128] for g in range(4)]
                          + [b2r[:, g, 128:] for g in range(4)], axis=-1)
    B2p = jnp.transpose(b2p)
    W3t = jnp.transpose(w3).astype(jnp.bfloat16)
    W4t = jnp.transpose(w4).astype(jnp.bfloat16)
    W5t = jnp.transpose(w5).astype(jnp.bfloat16)
    b1t = jnp.transpose(b1)
    b3t = jnp.transpose(b3)
    b4t = jnp.transpose(b4)
    b5t = jnp.transpose(b5)

    out = pl.pallas_call(
        _net1_body,
        out_shape=jax.ShapeDtypeStruct((nb, block_b, 128), jnp.float32),
        grid=(nb,),
        in_specs=[
            pl.BlockSpec((28, 28, 1, block_b), lambda i: (0, 0, 0, i)),
            pl.BlockSpec((1024, 200), lambda i: (0, 0)),
            pl.BlockSpec((128, 1), lambda i: (0, 0)),
            pl.BlockSpec((640, 1280), lambda i: (0, 0)),
            pl.BlockSpec((640, 1), lambda i: (0, 0)),
            pl.BlockSpec((64, 800), lambda i: (0, 0)),
            pl.BlockSpec((64, 1), lambda i: (0, 0)),
            pl.BlockSpec((32, 64), lambda i: (0, 0)),
            pl.BlockSpec((32, 1), lambda i: (0, 0)),
            pl.BlockSpec((128, 32), lambda i: (0, 0)),
            pl.BlockSpec((128, 1), lambda i: (0, 0)),
        ],
        out_specs=pl.BlockSpec((1, block_b, 128), lambda i: (i, 0, 0)),
        compiler_params=pltpu.CompilerParams(
            dimension_semantics=("parallel",),
            vmem_limit_bytes=64 * 1024 * 1024),
    )(xt, W1p, b1t, W2p, B2p, W3t, b3t, W4t, b4t, W5t, b5t)
    return out.reshape(B, 128)[:, :10]


def kernel(x_nchw, w1, b1, w2, b2, w3, b3, w4, b4, w5, b5):
    B = x_nchw.shape[0]
    block_b = 128 if B % 128 == 0 else (32 if B % 32 == 0 else 1)
    return _net1_forward(x_nchw, w1, b1, w2, b2, w3, b3, w4, b4, w5, b5,
                         block_b=block_b)


# speedup vs baseline: 4.7482x; 1.0130x over previous
import functools

import jax
import jax.numpy as jnp
from jax.experimental import pallas as pl
from jax.experimental.pallas import tpu as pltpu


def _net1_body(x_ref, w1_ref, b1_ref, w2_ref, b2_ref, w3_ref, b3_ref,
               w4_ref, b4_ref, w5_ref, b5_ref, o_ref):
    bt = x_ref.shape[3]
    f32 = jnp.float32
    bf16 = jnp.bfloat16

    x = x_ref[...].reshape(28, 28, bt).astype(bf16)
    xp = jnp.concatenate([x, jnp.zeros((28, 4, bt), bf16)], axis=1)
    xp = jnp.concatenate([xp, jnp.zeros((3, 32, bt), bf16)], axis=0)

    p1 = jnp.concatenate(
        [xp[4 * ar + 2 * pr:4 * ar + 2 * pr + 5].reshape(160, bt)
         for pr in range(2) for ar in range(7)], axis=-1)

    h1 = jnp.dot(w1_ref[...], p1, preferred_element_type=f32)
    b1v = b1_ref[...]
    y1 = []
    for ps in range(2):
        o = ps * 512
        m = jnp.maximum(jnp.maximum(h1[o:o + 128], h1[o + 128:o + 256]),
                        jnp.maximum(h1[o + 256:o + 384], h1[o + 384:o + 512]))
        y1.append(jnp.maximum(m + b1v, 0.0).astype(bf16))

    pieces = []
    for Dr in range(5):
        t = Dr // 2
        pr = Dr % 2
        for ps in range(2):
            o = pr * 7 * bt + t * bt
            pieces.append(y1[ps][:, o:o + 5 * bt])
    p2 = jnp.concatenate(pieces, axis=0)

    h2 = jnp.dot(w2_ref[...], p2, preferred_element_type=f32)
    h2 = jnp.maximum(h2 + b2_ref[...], 0.0)
    y2a = (h2[0:128] + h2[128:256]) + (h2[256:384] + h2[384:512])
    y2b = (h2[512:544] + h2[544:576]) + (h2[576:608] + h2[608:640])
    y2 = (0.25 * jnp.concatenate([y2a, y2b], axis=0)).astype(bf16)

    p3 = jnp.concatenate([y2[:, R * bt:(R + 1) * bt] for R in range(5)],
                         axis=0)
    h3 = jnp.dot(w3_ref[...], p3, preferred_element_type=f32)
    h3 = jnp.maximum(h3 + b3_ref[...], 0.0).astype(bf16)

    h4 = jnp.tanh(jnp.dot(w4_ref[...], h3, preferred_element_type=f32)
                  + b4_ref[...]).astype(bf16)
    lg = jnp.dot(w5_ref[...], h4, preferred_element_type=f32) + b5_ref[...]
    m = jnp.max(lg, axis=0, keepdims=True)
    e = jnp.exp(lg - m)
    s = jnp.sum(e, axis=0, keepdims=True)
    o_ref[...] = jnp.transpose(e / s)[None]


@functools.partial(jax.jit, static_argnames=("block_b",))
def _net1_forward(x_nchw, w1, b1, w2, b2, w3, b3, w4, b4, w5, b5, block_b=128):
    B = x_nchw.shape[0]
    nb = B // block_b
    xt = jnp.transpose(x_nchw, (2, 3, 1, 0))

    w1r = w1.reshape(5, 32, 512)
    W1p = jnp.stack([w1r, jnp.pad(w1r[:, :30], ((0, 0), (2, 0), (0, 0)))],
                    axis=0)
    W1p = jnp.transpose(W1p, (0, 3, 1, 2)).reshape(1024, 160)
    W1p = W1p.astype(jnp.bfloat16)

    w2r = w2.reshape(1280, 4, 160)
    w2p = jnp.concatenate([w2r[:, g, :128] for g in range(4)]
                          + [w2r[:, g, 128:] for g in range(4)], axis=-1)
    W2p = jnp.transpose(w2p).astype(jnp.bfloat16)
    b2r = b2.reshape(1, 4, 160)
    b2p = jnp.concatenate([b2r[:, g, :128] for g in range(4)]
                          + [b2r[:, g, 128:] for g in range(4)], axis=-1)
    B2p = jnp.transpose(b2p)
    W3t = jnp.transpose(w3).astype(jnp.bfloat16)
    W4t = jnp.transpose(w4).astype(jnp.bfloat16)
    W5t = jnp.transpose(w5).astype(jnp.bfloat16)
    b1t = jnp.transpose(b1)
    b3t = jnp.transpose(b3)
    b4t = jnp.transpose(b4)
    b5t = jnp.transpose(b5)

    out = pl.pallas_call(
        _net1_body,
        out_shape=jax.ShapeDtypeStruct((nb, block_b, 128), jnp.float32),
        grid=(nb,),
        in_specs=[
            pl.BlockSpec((28, 28, 1, block_b), lambda i: (0, 0, 0, i)),
            pl.BlockSpec((1024, 160), lambda i: (0, 0)),
            pl.BlockSpec((128, 1), lambda i: (0, 0)),
            pl.BlockSpec((640, 1280), lambda i: (0, 0)),
            pl.BlockSpec((640, 1), lambda i: (0, 0)),
            pl.BlockSpec((64, 800), lambda i: (0, 0)),
            pl.BlockSpec((64, 1), lambda i: (0, 0)),
            pl.BlockSpec((32, 64), lambda i: (0, 0)),
            pl.BlockSpec((32, 1), lambda i: (0, 0)),
            pl.BlockSpec((128, 32), lambda i: (0, 0)),
            pl.BlockSpec((128, 1), lambda i: (0, 0)),
        ],
        out_specs=pl.BlockSpec((1, block_b, 128), lambda i: (i, 0, 0)),
        compiler_params=pltpu.CompilerParams(
            dimension_semantics=("parallel",),
            vmem_limit_bytes=64 * 1024 * 1024),
    )(xt, W1p, b1t, W2p, B2p, W3t, b3t, W4t, b4t, W5t, b5t)
    return out.reshape(B, 128)[:, :10]


def kernel(x_nchw, w1, b1, w2, b2, w3, b3, w4, b4, w5, b5):
    B = x_nchw.shape[0]
    block_b = 128 if B % 128 == 0 else (32 if B % 32 == 0 else 1)
    return _net1_forward(x_nchw, w1, b1, w2, b2, w3, b3, w4, b4, w5, b5,
                         block_b=block_b)


# speedup vs baseline: 6.6685x; 1.4044x over previous
import functools

import jax
import jax.numpy as jnp
from jax.experimental import pallas as pl
from jax.experimental.pallas import tpu as pltpu


def _net1_body(x_ref, w1_ref, b1_ref, w2_ref, b2_ref, w3_ref, b3_ref,
               w4_ref, b4_ref, w5_ref, b5_ref, o_ref):
    bt = x_ref.shape[3]
    f32 = jnp.float32
    bf16 = jnp.bfloat16

    x = x_ref[...].reshape(28, 28, bt).astype(bf16)
    xp = jnp.concatenate([x, jnp.zeros((28, 4, bt), bf16)], axis=1)
    xp = jnp.concatenate([xp, jnp.zeros((3, 32, bt), bf16)], axis=0)

    p1 = jnp.concatenate(
        [xp[4 * ar + 2 * pr:4 * ar + 2 * pr + 5].reshape(160, bt)
         for pr in range(2) for ar in range(7)], axis=-1)

    h1 = jnp.dot(w1_ref[...], p1,
                 preferred_element_type=f32).astype(bf16)
    b1v = b1_ref[...].astype(bf16)
    y1 = []
    for ps in range(2):
        o = ps * 512
        m = jnp.maximum(jnp.maximum(h1[o:o + 128], h1[o + 128:o + 256]),
                        jnp.maximum(h1[o + 256:o + 384], h1[o + 384:o + 512]))
        y1.append(jnp.maximum(m + b1v, 0.0))

    pieces = []
    for Dr in range(5):
        t = Dr // 2
        pr = Dr % 2
        for ps in range(2):
            o = pr * 7 * bt + t * bt
            pieces.append(y1[ps][:, o:o + 5 * bt])
    p2 = jnp.concatenate(pieces, axis=0)

    h2 = jnp.dot(w2_ref[...], p2, preferred_element_type=f32)
    h2 = jnp.maximum(h2 + b2_ref[...], 0.0).astype(bf16)
    y2a = (h2[0:128] + h2[128:256]) + (h2[256:384] + h2[384:512])
    y2b = (h2[512:544] + h2[544:576]) + (h2[576:608] + h2[608:640])
    y2 = jnp.float32(0.25).astype(bf16) * jnp.concatenate([y2a, y2b], axis=0)

    p3 = jnp.concatenate([y2[:, R * bt:(R + 1) * bt] for R in range(5)],
                         axis=0)
    h3 = jnp.dot(w3_ref[...], p3, preferred_element_type=f32)
    h3 = jnp.maximum(h3 + b3_ref[...], 0.0).astype(bf16)

    h4 = jnp.tanh(jnp.dot(w4_ref[...], h3, preferred_element_type=f32)
                  + b4_ref[...]).astype(bf16)
    lg = jnp.dot(w5_ref[...], h4, preferred_element_type=f32) + b5_ref[...]
    m = jnp.max(lg, axis=0, keepdims=True)
    e = jnp.exp(lg - m)
    s = jnp.sum(e, axis=0, keepdims=True)
    o_ref[...] = jnp.transpose(e / s)[None]


@functools.partial(jax.jit, static_argnames=("block_b",))
def _net1_forward(x_nchw, w1, b1, w2, b2, w3, b3, w4, b4, w5, b5, block_b=128):
    B = x_nchw.shape[0]
    nb = B // block_b
    xt = jnp.transpose(x_nchw, (2, 3, 1, 0))

    w1r = w1.reshape(5, 32, 512)
    W1p = jnp.stack([w1r, jnp.pad(w1r[:, :30], ((0, 0), (2, 0), (0, 0)))],
                    axis=0)
    W1p = jnp.transpose(W1p, (0, 3, 1, 2)).reshape(1024, 160)
    W1p = W1p.astype(jnp.bfloat16)

    w2r = w2.reshape(1280, 4, 160)
    w2p = jnp.concatenate([w2r[:, g, :128] for g in range(4)]
                          + [w2r[:, g, 128:] for g in range(4)], axis=-1)
    W2p = jnp.transpose(w2p).astype(jnp.bfloat16)
    b2r = b2.reshape(1, 4, 160)
    b2p = jnp.concatenate([b2r[:, g, :128] for g in range(4)]
                          + [b2r[:, g, 128:] for g in range(4)], axis=-1)
    B2p = jnp.transpose(b2p)
    W3t = jnp.transpose(w3).astype(jnp.bfloat16)
    W4t = jnp.transpose(w4).astype(jnp.bfloat16)
    W5t = jnp.transpose(w5).astype(jnp.bfloat16)
    b1t = jnp.transpose(b1)
    b3t = jnp.transpose(b3)
    b4t = jnp.transpose(b4)
    b5t = jnp.transpose(b5)

    out = pl.pallas_call(
        _net1_body,
        out_shape=jax.ShapeDtypeStruct((nb, block_b, 128), jnp.float32),
        grid=(nb,),
        in_specs=[
            pl.BlockSpec((28, 28, 1, block_b), lambda i: (0, 0, 0, i)),
            pl.BlockSpec((1024, 160), lambda i: (0, 0)),
            pl.BlockSpec((128, 1), lambda i: (0, 0)),
            pl.BlockSpec((640, 1280), lambda i: (0, 0)),
            pl.BlockSpec((640, 1), lambda i: (0, 0)),
            pl.BlockSpec((64, 800), lambda i: (0, 0)),
            pl.BlockSpec((64, 1), lambda i: (0, 0)),
            pl.BlockSpec((32, 64), lambda i: (0, 0)),
            pl.BlockSpec((32, 1), lambda i: (0, 0)),
            pl.BlockSpec((128, 32), lambda i: (0, 0)),
            pl.BlockSpec((128, 1), lambda i: (0, 0)),
        ],
        out_specs=pl.BlockSpec((1, block_b, 128), lambda i: (i, 0, 0)),
        compiler_params=pltpu.CompilerParams(
            dimension_semantics=("parallel",),
            vmem_limit_bytes=64 * 1024 * 1024),
    )(xt, W1p, b1t, W2p, B2p, W3t, b3t, W4t, b4t, W5t, b5t)
    return out.reshape(B, 128)[:, :10]


def kernel(x_nchw, w1, b1, w2, b2, w3, b3, w4, b4, w5, b5):
    B = x_nchw.shape[0]
    block_b = 256 if B % 256 == 0 else (32 if B % 32 == 0 else 1)
    return _net1_forward(x_nchw, w1, b1, w2, b2, w3, b3, w4, b4, w5, b5,
                         block_b=block_b)


# speedup vs baseline: 7.4257x; 1.1135x over previous
import functools

import jax
import jax.numpy as jnp
from jax.experimental import pallas as pl
from jax.experimental.pallas import tpu as pltpu


def _net1_body(x_ref, w1_ref, b1_ref, w2_ref, b2_ref, w3_ref, b3_ref,
               w4_ref, b4_ref, w5_ref, b5_ref, o_ref):
    bt = x_ref.shape[3]
    f32 = jnp.float32
    bf16 = jnp.bfloat16

    x = x_ref[...].reshape(28, 28, bt).astype(bf16)
    xp = jnp.concatenate([x, jnp.zeros((28, 4, bt), bf16)], axis=1)
    xp = jnp.concatenate([xp, jnp.zeros((3, 32, bt), bf16)], axis=0)

    p1 = jnp.concatenate(
        [xp[4 * ar + 2 * pr:4 * ar + 2 * pr + 5].reshape(160, bt)
         for pr in range(2) for ar in range(7)], axis=-1)

    h1 = jnp.dot(w1_ref[...], p1,
                 preferred_element_type=f32).astype(bf16)
    b1v = b1_ref[...].astype(bf16)
    y1 = []
    for ps in range(2):
        o = ps * 512
        m = jnp.maximum(jnp.maximum(h1[o:o + 128], h1[o + 128:o + 256]),
                        jnp.maximum(h1[o + 256:o + 384], h1[o + 384:o + 512]))
        y1.append(jnp.maximum(m + b1v, 0.0))

    pieces = []
    for Dr in range(5):
        t = Dr // 2
        pr = Dr % 2
        for ps in range(2):
            o = pr * 7 * bt + t * bt
            pieces.append(y1[ps][:, o:o + 5 * bt])
    p2 = jnp.concatenate(pieces, axis=0)

    h2 = jnp.dot(w2_ref[...], p2, preferred_element_type=f32)
    h2 = jnp.maximum(h2 + b2_ref[...], 0.0).astype(bf16)
    y2a = (h2[0:128] + h2[128:256]) + (h2[256:384] + h2[384:512])
    y2b = (h2[512:544] + h2[544:576]) + (h2[576:608] + h2[608:640])
    y2 = jnp.float32(0.25).astype(bf16) * jnp.concatenate([y2a, y2b], axis=0)

    p3 = jnp.concatenate([y2[:, R * bt:(R + 1) * bt] for R in range(5)],
                         axis=0)
    h3 = jnp.dot(w3_ref[...], p3, preferred_element_type=f32)
    h3 = jnp.maximum(h3 + b3_ref[...], 0.0).astype(bf16)

    h4 = jnp.tanh(jnp.dot(w4_ref[...], h3, preferred_element_type=f32)
                  + b4_ref[...]).astype(bf16)
    lg = jnp.dot(w5_ref[...], h4, preferred_element_type=f32) + b5_ref[...]
    m = jnp.max(lg, axis=0, keepdims=True)
    e = jnp.exp(lg - m)
    s = jnp.sum(e, axis=0, keepdims=True)
    o_ref[...] = jnp.transpose(e / s)[None]


@functools.partial(jax.jit, static_argnames=("block_b",))
def _net1_forward(x_nchw, w1, b1, w2, b2, w3, b3, w4, b4, w5, b5, block_b=128):
    B = x_nchw.shape[0]
    nb = B // block_b
    xt = jnp.transpose(x_nchw, (2, 3, 1, 0))

    w1r = w1.reshape(5, 32, 512)
    W1p = jnp.stack([w1r, jnp.pad(w1r[:, :30], ((0, 0), (2, 0), (0, 0)))],
                    axis=0)
    W1p = jnp.transpose(W1p, (0, 3, 1, 2)).reshape(1024, 160)
    W1p = W1p.astype(jnp.bfloat16)

    w2r = w2.reshape(1280, 4, 160)
    w2p = jnp.concatenate([w2r[:, g, :128] for g in range(4)]
                          + [w2r[:, g, 128:] for g in range(4)], axis=-1)
    W2p = jnp.transpose(w2p).astype(jnp.bfloat16)
    b2r = b2.reshape(1, 4, 160)
    b2p = jnp.concatenate([b2r[:, g, :128] for g in range(4)]
                          + [b2r[:, g, 128:] for g in range(4)], axis=-1)
    B2p = jnp.transpose(b2p)
    W3t = jnp.transpose(w3).astype(jnp.bfloat16)
    W4t = jnp.transpose(w4).astype(jnp.bfloat16)
    W5t = jnp.transpose(w5).astype(jnp.bfloat16)
    b1t = jnp.transpose(b1)
    b3t = jnp.transpose(b3)
    b4t = jnp.transpose(b4)
    b5t = jnp.transpose(b5)

    out = pl.pallas_call(
        _net1_body,
        out_shape=jax.ShapeDtypeStruct((nb, block_b, 128), jnp.float32),
        grid=(nb,),
        in_specs=[
            pl.BlockSpec((28, 28, 1, block_b), lambda i: (0, 0, 0, i)),
            pl.BlockSpec((1024, 160), lambda i: (0, 0)),
            pl.BlockSpec((128, 1), lambda i: (0, 0)),
            pl.BlockSpec((640, 1280), lambda i: (0, 0)),
            pl.BlockSpec((640, 1), lambda i: (0, 0)),
            pl.BlockSpec((64, 800), lambda i: (0, 0)),
            pl.BlockSpec((64, 1), lambda i: (0, 0)),
            pl.BlockSpec((32, 64), lambda i: (0, 0)),
            pl.BlockSpec((32, 1), lambda i: (0, 0)),
            pl.BlockSpec((128, 32), lambda i: (0, 0)),
            pl.BlockSpec((128, 1), lambda i: (0, 0)),
        ],
        out_specs=pl.BlockSpec((1, block_b, 128), lambda i: (i, 0, 0)),
        compiler_params=pltpu.CompilerParams(
            dimension_semantics=("parallel",),
            vmem_limit_bytes=64 * 1024 * 1024),
    )(xt, W1p, b1t, W2p, B2p, W3t, b3t, W4t, b4t, W5t, b5t)
    return out.reshape(B, 128)[:, :10]


def kernel(x_nchw, w1, b1, w2, b2, w3, b3, w4, b4, w5, b5):
    B = x_nchw.shape[0]
    block_b = 512 if B % 512 == 0 else (32 if B % 32 == 0 else 1)
    return _net1_forward(x_nchw, w1, b1, w2, b2, w3, b3, w4, b4, w5, b5,
                         block_b=block_b)


# speedup vs baseline: 7.4934x; 1.0091x over previous
import functools

import jax
import jax.numpy as jnp
from jax.experimental import pallas as pl
from jax.experimental.pallas import tpu as pltpu


def _net1_body(x_ref, w1_ref, b1_ref, w2_ref, b2_ref, w3_ref, b3_ref,
               w4_ref, b4_ref, w5_ref, b5_ref, o_ref):
    bt = x_ref.shape[3]
    f32 = jnp.float32
    bf16 = jnp.bfloat16

    x = x_ref[...].reshape(28, 28, bt).astype(bf16)
    xp = jnp.concatenate([x, jnp.zeros((28, 4, bt), bf16)], axis=1)
    xp = jnp.concatenate([xp, jnp.zeros((3, 32, bt), bf16)], axis=0)

    p1 = jnp.concatenate(
        [xp[4 * ar + 2 * pr:4 * ar + 2 * pr + 5].reshape(160, bt)
         for pr in range(2) for ar in range(7)], axis=-1)

    h1 = jnp.dot(w1_ref[...], p1,
                 preferred_element_type=f32).astype(bf16)
    b1v = b1_ref[...].astype(bf16)
    y1 = []
    for ps in range(2):
        o = ps * 512
        m = jnp.maximum(jnp.maximum(h1[o:o + 128], h1[o + 128:o + 256]),
                        jnp.maximum(h1[o + 256:o + 384], h1[o + 384:o + 512]))
        y1.append(jnp.maximum(m + b1v, 0.0))

    pieces = []
    for Dr in range(5):
        t = Dr // 2
        pr = Dr % 2
        for ps in range(2):
            o = pr * 7 * bt + t * bt
            pieces.append(y1[ps][:, o:o + 5 * bt])
    p2 = jnp.concatenate(pieces, axis=0)

    h2 = jnp.dot(w2_ref[...], p2, preferred_element_type=f32)
    h2 = jnp.maximum(h2 + b2_ref[...], 0.0).astype(bf16)
    y2a = (h2[0:128] + h2[128:256]) + (h2[256:384] + h2[384:512])
    y2b = (h2[512:544] + h2[544:576]) + (h2[576:608] + h2[608:640])
    y2 = jnp.float32(0.25).astype(bf16) * jnp.concatenate([y2a, y2b], axis=0)

    p3 = jnp.concatenate([y2[:, R * bt:(R + 1) * bt] for R in range(5)],
                         axis=0)
    h3 = jnp.dot(w3_ref[...], p3, preferred_element_type=f32)
    h3 = jnp.maximum(h3 + b3_ref[...], 0.0).astype(bf16)

    h4 = jnp.tanh(jnp.dot(w4_ref[...], h3, preferred_element_type=f32)
                  + b4_ref[...]).astype(bf16)
    lg = jnp.dot(w5_ref[...], h4, preferred_element_type=f32) + b5_ref[...]
    m = jnp.max(lg, axis=0, keepdims=True)
    e = jnp.exp(lg - m)
    s = jnp.sum(e, axis=0, keepdims=True)
    o_ref[...] = jnp.transpose(e / s)[None]


@functools.partial(jax.jit, static_argnames=("block_b",))
def _net1_forward(x_nchw, w1, b1, w2, b2, w3, b3, w4, b4, w5, b5, block_b=128):
    B = x_nchw.shape[0]
    nb = B // block_b
    xt = jnp.transpose(x_nchw, (2, 3, 1, 0))

    w1r = w1.reshape(5, 32, 512)
    W1p = jnp.stack([w1r, jnp.pad(w1r[:, :30], ((0, 0), (2, 0), (0, 0)))],
                    axis=0)
    W1p = jnp.transpose(W1p, (0, 3, 1, 2)).reshape(1024, 160)
    W1p = W1p.astype(jnp.bfloat16)

    w2r = w2.reshape(1280, 4, 160)
    w2p = jnp.concatenate([w2r[:, g, :128] for g in range(4)]
                          + [w2r[:, g, 128:] for g in range(4)], axis=-1)
    W2p = jnp.transpose(w2p).astype(jnp.bfloat16)
    b2r = b2.reshape(1, 4, 160)
    b2p = jnp.concatenate([b2r[:, g, :128] for g in range(4)]
                          + [b2r[:, g, 128:] for g in range(4)], axis=-1)
    B2p = jnp.transpose(b2p)
    W3t = jnp.transpose(w3).astype(jnp.bfloat16)
    W4t = jnp.transpose(w4).astype(jnp.bfloat16)
    W5t = jnp.transpose(w5).astype(jnp.bfloat16)
    b1t = jnp.transpose(b1)
    b3t = jnp.transpose(b3)
    b4t = jnp.transpose(b4)
    b5t = jnp.transpose(b5)

    out = pl.pallas_call(
        _net1_body,
        out_shape=jax.ShapeDtypeStruct((nb, block_b, 128), jnp.float32),
        grid=(nb,),
        in_specs=[
            pl.BlockSpec((28, 28, 1, block_b), lambda i: (0, 0, 0, i)),
            pl.BlockSpec((1024, 160), lambda i: (0, 0)),
            pl.BlockSpec((128, 1), lambda i: (0, 0)),
            pl.BlockSpec((640, 1280), lambda i: (0, 0)),
            pl.BlockSpec((640, 1), lambda i: (0, 0)),
            pl.BlockSpec((64, 800), lambda i: (0, 0)),
            pl.BlockSpec((64, 1), lambda i: (0, 0)),
            pl.BlockSpec((32, 64), lambda i: (0, 0)),
            pl.BlockSpec((32, 1), lambda i: (0, 0)),
            pl.BlockSpec((128, 32), lambda i: (0, 0)),
            pl.BlockSpec((128, 1), lambda i: (0, 0)),
        ],
        out_specs=pl.BlockSpec((1, block_b, 128), lambda i: (i, 0, 0)),
        compiler_params=pltpu.CompilerParams(
            dimension_semantics=("parallel",),
            vmem_limit_bytes=100 * 1024 * 1024),
    )(xt, W1p, b1t, W2p, B2p, W3t, b3t, W4t, b4t, W5t, b5t)
    return out.reshape(B, 128)[:, :10]


def kernel(x_nchw, w1, b1, w2, b2, w3, b3, w4, b4, w5, b5):
    B = x_nchw.shape[0]
    block_b = 1024 if B % 1024 == 0 else (32 if B % 32 == 0 else 1)
    return _net1_forward(x_nchw, w1, b1, w2, b2, w3, b3, w4, b4, w5, b5,
                         block_b=block_b)
